# Initial kernel scaffold; baseline (speedup 1.0000x reference)
import jax
import jax.numpy as jnp
from jax.experimental import pallas as pl


def kernel(x, batch):
    raise NotImplementedError("write your pallas kernel here")


# speedup vs baseline: 4.2446x; 4.2446x over previous
import functools
import jax
import jax.numpy as jnp
from jax import lax
from jax.experimental import pallas as pl
from jax.experimental.pallas import tpu as pltpu, tpu_sc as plsc

NUM_NODES = 50000
D_FEAT = 256
NUM_GRAPHS = 128

NUM_CORES = 2
NUM_SUBCORES = 16
COLS_PER_CORE = D_FEAT // NUM_CORES

CHUNK = 80
NUM_CHUNKS = NUM_NODES // CHUNK
MAX_ITERS = NUM_CHUNKS // NUM_SUBCORES + 1


def _pool_kernel(x_hbm, batch_hbm, out_hbm, idx_v, rows_v, obuf_v, acc_sh):
    c = lax.axis_index("c")
    s = lax.axis_index("s")
    col0 = c * COLS_PER_CORE

    zeros16 = jnp.zeros((16,), jnp.float32)
    for i in range(8):
        for j in range(COLS_PER_CORE // 16):
            obuf_v[i, pl.ds(j * 16, 16)] = zeros16
    pltpu.sync_copy(obuf_v, acc_sh.at[pl.ds(s * 8, 8), :])
    plsc.subcore_barrier()

    extra = jnp.where(s > 0, 1, 0)
    start = s * (MAX_ITERS - 1) + extra
    count = MAX_ITERS - extra

    def body(j, carry):
        @pl.when(j < count)
        def _():
            r0 = (start + j) * CHUNK
            pltpu.sync_copy(batch_hbm.at[pl.ds(r0, CHUNK)], idx_v)
            pltpu.sync_copy(
                x_hbm.at[pl.ds(r0, CHUNK), pl.ds(col0, COLS_PER_CORE)], rows_v
            )
            pltpu.sync_copy(rows_v, acc_sh.at[idx_v], add=True)

        return carry

    lax.fori_loop(0, MAX_ITERS, body, 0)
    plsc.subcore_barrier()

    pltpu.sync_copy(acc_sh.at[pl.ds(s * 8, 8), :], obuf_v)
    pltpu.sync_copy(
        obuf_v, out_hbm.at[pl.ds(s * 8, 8), pl.ds(col0, COLS_PER_CORE)]
    )


@jax.jit
def kernel(x, batch):
    batch = batch.astype(jnp.int32)
    mesh = plsc.VectorSubcoreMesh(core_axis_name="c", subcore_axis_name="s")
    return pl.kernel(
        _pool_kernel,
        out_type=jax.ShapeDtypeStruct((NUM_GRAPHS, D_FEAT), jnp.float32),
        mesh=mesh,
        scratch_types=[
            pltpu.VMEM((CHUNK,), jnp.int32),
            pltpu.VMEM((CHUNK, COLS_PER_CORE), jnp.float32),
            pltpu.VMEM((8, COLS_PER_CORE), jnp.float32),
            pltpu.VMEM_SHARED((NUM_GRAPHS, COLS_PER_CORE), jnp.float32),
        ],
    )(x, batch)


# speedup vs baseline: 7.6766x; 1.8086x over previous
import jax
import jax.numpy as jnp
from jax import lax
from jax.experimental import pallas as pl
from jax.experimental.pallas import tpu as pltpu, tpu_sc as plsc

NUM_NODES = 50000
D_FEAT = 256
NUM_GRAPHS = 128

NUM_CORES = 2
NUM_SUBCORES = 16
COLS_PER_CORE = D_FEAT // NUM_CORES

CHUNK = 80
NUM_CHUNKS = NUM_NODES // CHUNK
MAX_ITERS = -(-NUM_CHUNKS // NUM_SUBCORES)
PAD_CHUNKS = MAX_ITERS * NUM_SUBCORES


def _pool_kernel(x_hbm, batch3d_hbm, out_hbm,
                 idx2d_v, rows_v, obuf_v, acc_sh, sem0, sem1):
    c = lax.axis_index("c")
    s = lax.axis_index("s")
    col0 = c * COLS_PER_CORE
    sems = (sem0, sem1)

    zeros16 = jnp.zeros((16,), jnp.float32)
    for i in range(8):
        for j in range(COLS_PER_CORE // 16):
            obuf_v[i, pl.ds(j * 16, 16)] = zeros16
    pltpu.sync_copy(obuf_v, acc_sh.at[pl.ds(s * 8, 8), :])
    plsc.subcore_barrier()

    start = s * MAX_ITERS
    count = jnp.minimum(MAX_ITERS, NUM_CHUNKS - start)

    pltpu.sync_copy(batch3d_hbm.at[s], idx2d_v)

    def row_src(j):
        return x_hbm.at[pl.ds((start + j) * CHUNK, CHUNK),
                        pl.ds(col0, COLS_PER_CORE)]

    pltpu.async_copy(row_src(0), rows_v.at[0], sem0)

    def body(i, carry):
        for b in range(2):
            j = 2 * i + b
            nb = (b + 1) % 2

            @pl.when(j + 1 < count)
            def _():
                pltpu.async_copy(row_src(j + 1), rows_v.at[nb], sems[nb])

            @pl.when(j < count)
            def _():
                pltpu.make_async_copy(row_src(0), rows_v.at[b],
                                      sems[b]).wait()
                pltpu.sync_copy(rows_v.at[b], acc_sh.at[idx2d_v.at[j]],
                                add=True)

        return carry

    lax.fori_loop(0, MAX_ITERS // 2, body, 0)
    plsc.subcore_barrier()

    pltpu.sync_copy(acc_sh.at[pl.ds(s * 8, 8), :], obuf_v)
    pltpu.sync_copy(
        obuf_v, out_hbm.at[pl.ds(s * 8, 8), pl.ds(col0, COLS_PER_CORE)]
    )


@jax.jit
def kernel(x, batch):
    batch3d = jnp.pad(
        batch.astype(jnp.int32), (0, PAD_CHUNKS * CHUNK - NUM_NODES)
    ).reshape(NUM_SUBCORES, MAX_ITERS, CHUNK)
    mesh = plsc.VectorSubcoreMesh(core_axis_name="c", subcore_axis_name="s")
    return pl.kernel(
        _pool_kernel,
        out_type=jax.ShapeDtypeStruct((NUM_GRAPHS, D_FEAT), jnp.float32),
        mesh=mesh,
        scratch_types=[
            pltpu.VMEM((MAX_ITERS, CHUNK), jnp.int32),
            pltpu.VMEM((2, CHUNK, COLS_PER_CORE), jnp.float32),
            pltpu.VMEM((8, COLS_PER_CORE), jnp.float32),
            pltpu.VMEM_SHARED((NUM_GRAPHS, COLS_PER_CORE), jnp.float32),
            pltpu.SemaphoreType.DMA,
            pltpu.SemaphoreType.DMA,
        ],
    )(x, batch3d)
